# baseline (speedup 1.0000x reference)
import functools

import jax
import jax.numpy as jnp
from jax import lax
from jax.experimental import pallas as pl
from jax.experimental.pallas import tpu as pltpu
from jax.experimental.pallas import tpu_sc as plsc

N = 10000
E = 320000
D = 128
ED = 16
EPS = 1e-5
HI = lax.Precision.HIGHEST
F32 = jnp.float32

K = 64
KC = 128
NCHUNK = E // K
NCHUNKC = E // KC
NW = 32
TMAX = -(-NCHUNK // NW)
TMAXC = -(-NCHUNKC // NW)
NT = 16
NPAD = 10240
SPAD = 10112
SRPT = SPAD // NT
SQ = (64,) * 9 + (56,)


def _dot(a, b):
    return jnp.dot(a, b, preferred_element_type=F32, precision=HI)


def _ln(h, g, b):
    m = jnp.mean(h, axis=-1, keepdims=True)
    c = h - m
    v = jnp.mean(c * c, axis=-1, keepdims=True)
    return c * lax.rsqrt(v + EPS) * g + b


BN = 1000

def _enc_body(x_ref, ewt, eb, eg, ebeta, wdt, wst, h_ref, ad_ref, as_ref):
    h = jnp.maximum(_dot(x_ref[...], ewt[...]) + eb[...], 0.0)
    h = _ln(h, eg[...], ebeta[...])
    h_ref[...] = h
    ad_ref[...] = _dot(h, wdt[...])
    as_ref[...] = _dot(h, wst[...])


_full = lambda shape: pl.BlockSpec(shape, lambda i: (0,) * len(shape))
_rows = lambda shape: pl.BlockSpec(shape, lambda i: (i,) + (0,) * (len(shape) - 1))

_enc_call = pl.pallas_call(
    _enc_body,
    grid=(N // BN,),
    in_specs=[_rows((BN, D)), _full((D, D)), _full((1, D)), _full((1, D)),
              _full((1, D)), _full((D, D)), _full((D, D))],
    out_specs=[_rows((BN, D))] * 3,
    out_shape=[jax.ShapeDtypeStruct((N, D), F32)] * 3,
)

BE = 2000

def _c_body(ea_ref, w0t, b0, w1t, b1, c0_ref, c1_ref):
    ea = ea_ref[...]
    c0_ref[...] = _dot(ea, w0t[...]) + b0[...]
    c1_ref[...] = _dot(ea, w1t[...]) + b1[...]


_c_call = pl.pallas_call(
    _c_body,
    grid=(E // BE,),
    in_specs=[_rows((BE, ED)), _full((ED, D)), _full((1, D)),
              _full((ED, D)), _full((1, D))],
    out_specs=[_rows((BE, D))] * 2,
    out_shape=[jax.ShapeDtypeStruct((E, D), F32)] * 2,
)

def _aggr_update(h, sp, cntp, w2t, mb2, u1at, u1bt, ub1, uw2t, ub2, lg, lb):
    s = sp[0] + sp[1]
    cnt = jnp.sum(cntp, axis=0)
    aggr = _dot(s, w2t) / jnp.maximum(cnt, 1.0)
    aggr = aggr + jnp.where(cnt > 0, 1.0, 0.0) * mb2
    u = jnp.maximum(_dot(h, u1at) + _dot(aggr, u1bt) + ub1, 0.0)
    u = _dot(u, uw2t) + ub2
    return _ln(h + u, lg, lb)


def _upd0_body(h_ref, sp_ref, cnt_ref, w2t, mb2, u1at, u1bt, ub1, uw2t, ub2,
               lg, lb, wdt, wst, h1_ref, ad_ref, as_ref):
    hn = _aggr_update(h_ref[...], sp_ref[...], cnt_ref[...], w2t[...],
                      mb2[...], u1at[...], u1bt[...], ub1[...], uw2t[...],
                      ub2[...], lg[...], lb[...])
    h1_ref[...] = hn
    ad_ref[...] = _dot(hn, wdt[...])
    as_ref[...] = _dot(hn, wst[...])


def _upd1_body(h_ref, sp_ref, cnt_ref, w2t, mb2, u1at, u1bt, ub1, uw2t, ub2,
               lg, lb, hwt, hb, out_ref):
    hn = _aggr_update(h_ref[...], sp_ref[...], cnt_ref[...], w2t[...],
                      mb2[...], u1at[...], u1bt[...], ub1[...], uw2t[...],
                      ub2[...], lg[...], lb[...])
    out_ref[...] = _dot(hn, hwt[...]) + hb[...]


_upd_common_specs = [
    _rows((BN, D)),
    pl.BlockSpec((2, BN, D), lambda i: (0, i, 0)),
    pl.BlockSpec((NW, BN, 1), lambda i: (0, i, 0)),
    _full((D, D)), _full((1, D)),
    _full((D, D)), _full((D, D)), _full((1, D)),
    _full((D, D)), _full((1, D)),
    _full((1, D)), _full((1, D)),
]

_upd0_call = pl.pallas_call(
    _upd0_body,
    grid=(N // BN,),
    in_specs=_upd_common_specs + [_full((D, D)), _full((D, D))],
    out_specs=[_rows((BN, D))] * 3,
    out_shape=[jax.ShapeDtypeStruct((N, D), F32)] * 3,
)

_upd1_call = pl.pallas_call(
    _upd1_body,
    grid=(N // BN,),
    in_specs=_upd_common_specs + [_full((D, D)), _full((1, D))],
    out_specs=_rows((BN, D)),
    out_shape=jax.ShapeDtypeStruct((N, D), F32),
)


def _sc_cnt_body(dst_hbm, cnt_hbm, dsti, cntloc):
    cid = lax.axis_index("c")
    sid = lax.axis_index("s")
    wid = sid * 2 + cid

    zero16 = jnp.zeros((16,), F32)
    ones16 = jnp.ones((16,), F32)

    def zbody(j, carry):
        for q in range(NPAD // KC // 16):
            cntloc[pl.ds(j * (NPAD // KC) + q * 16, 16)] = zero16
        return carry

    lax.fori_loop(0, KC, zbody, 0)

    def body(t, carry):
        chunk = wid + t * NW

        @pl.when(chunk < NCHUNKC)
        def _():
            pltpu.sync_copy(dst_hbm.at[pl.ds(chunk * KC, KC)], dsti)
            for u in range(KC // 16):
                plsc.addupdate_scatter(cntloc, [dsti[pl.ds(u * 16, 16)]], ones16)
        return carry

    lax.fori_loop(0, TMAXC, body, 0)
    pltpu.sync_copy(cntloc, cnt_hbm.at[pl.ds(wid * NPAD, NPAD)])


_sc_cnt_call = functools.partial(
    pl.kernel,
    out_type=jax.ShapeDtypeStruct((NW * NPAD,), F32),
    mesh=plsc.VectorSubcoreMesh(core_axis_name="c", subcore_axis_name="s"),
    compiler_params=pltpu.CompilerParams(needs_layout_passes=False),
    scratch_types=[
        pltpu.VMEM((KC,), jnp.int32),
        pltpu.VMEM((NPAD,), F32),
    ],
)(_sc_cnt_body)


def _sc_edge_body(ad_hbm, as_hbm, c_hbm, dst_hbm, src_hbm, out_hbm,
                  dsti, srci, adg, asg, cbuf, s_sh, sema, sems, semc):
    cid = lax.axis_index("c")
    sid = lax.axis_index("s")
    wid = sid * 2 + cid

    zero16 = jnp.zeros((16,), F32)

    def zbody(j, carry):
        for g in range(D // 16):
            cbuf[0, j, pl.ds(g * 16, 16)] = zero16
        return carry

    lax.fori_loop(0, K, zbody, 0)
    off = 0
    for q in SQ:
        pltpu.sync_copy(cbuf.at[0, pl.ds(0, q)],
                        s_sh.at[pl.ds(sid * SRPT + off, q)])
        off += q
    plsc.subcore_barrier()

    def issue(t):
        chunk = wid + t * NW
        b = lax.rem(t, 2)

        @pl.when(chunk < NCHUNK)
        def _():
            e0 = chunk * K
            pltpu.sync_copy(dst_hbm.at[pl.ds(e0, K)], dsti.at[b])
            pltpu.sync_copy(src_hbm.at[pl.ds(e0, K)], srci.at[b])
            pltpu.async_copy(ad_hbm.at[dsti.at[b]], adg.at[b], sema.at[b])
            pltpu.async_copy(as_hbm.at[srci.at[b]], asg.at[b], sems.at[b])
            pltpu.async_copy(c_hbm.at[pl.ds(e0, K)], cbuf.at[b], semc.at[b])

    def process(t):
        chunk = wid + t * NW
        b = lax.rem(t, 2)

        @pl.when(chunk < NCHUNK)
        def _():
            pltpu.make_async_copy(ad_hbm.at[dsti.at[b]], adg.at[b], sema.at[b]).wait()
            pltpu.make_async_copy(as_hbm.at[srci.at[b]], asg.at[b], sems.at[b]).wait()
            pltpu.make_async_copy(c_hbm.at[pl.ds(0, K)], cbuf.at[b], semc.at[b]).wait()

            def jbody(j, c2):
                for g in range(D // 16):
                    s = pl.ds(g * 16, 16)
                    cbuf[b, j, s] = jnp.maximum(
                        cbuf[b, j, s] + adg[b, j, s] + asg[b, j, s], 0.0)
                return c2

            lax.fori_loop(0, K, jbody, 0)
            pltpu.sync_copy(cbuf.at[b], s_sh.at[dsti.at[b]], add=True)

    issue(0)

    def body(t, carry):
        issue(t + 1)
        process(t)
        return carry

    lax.fori_loop(0, TMAX, body, 0)
    plsc.subcore_barrier()

    off = 0
    for q in SQ:
        pltpu.sync_copy(s_sh.at[pl.ds(sid * SRPT + off, q)],
                        cbuf.at[0, pl.ds(0, q)])
        pltpu.sync_copy(cbuf.at[0, pl.ds(0, q)],
                        out_hbm.at[pl.ds(cid * SPAD + sid * SRPT + off, q)])
        off += q


_sc_edge_call = functools.partial(
    pl.kernel,
    out_type=jax.ShapeDtypeStruct((2 * SPAD, D), F32),
    mesh=plsc.VectorSubcoreMesh(core_axis_name="c", subcore_axis_name="s"),
    compiler_params=pltpu.CompilerParams(needs_layout_passes=False),
    scratch_types=[
        pltpu.VMEM((2, K), jnp.int32),
        pltpu.VMEM((2, K), jnp.int32),
        pltpu.VMEM((2, K, D), F32),
        pltpu.VMEM((2, K, D), F32),
        pltpu.VMEM((2, K, D), F32),
        pltpu.VMEM_SHARED((SPAD, D), F32),
        pltpu.SemaphoreType.DMA((2,)),
        pltpu.SemaphoreType.DMA((2,)),
        pltpu.SemaphoreType.DMA((2,)),
    ],
)(_sc_edge_body)


def kernel(x, edge_index, edge_attr, enc_w, enc_b, enc_g, enc_beta,
           l0_mw1, l0_mb1, l0_mw2, l0_mb2, l0_uw1, l0_ub1, l0_uw2, l0_ub2, l0_lg, l0_lb,
           l1_mw1, l1_mb1, l1_mw2, l1_mb2, l1_uw1, l1_ub1, l1_uw2, l1_ub2, l1_lg, l1_lb,
           head_w, head_b):
    src = edge_index[0]
    dst = edge_index[1]
    r = lambda v: v.reshape(1, D)

    wd0t = l0_mw1[:, :D].T
    ws0t = l0_mw1[:, D:2 * D].T
    we0t = l0_mw1[:, 2 * D:].T
    wd1t = l1_mw1[:, :D].T
    ws1t = l1_mw1[:, D:2 * D].T
    we1t = l1_mw1[:, 2 * D:].T

    h0, ad0, as0 = _enc_call(x, enc_w.T, r(enc_b), r(enc_g), r(enc_beta),
                             wd0t, ws0t)
    c0, c1 = _c_call(edge_attr, we0t, r(l0_mb1), we1t, r(l1_mb1))
    cnt = _sc_cnt_call(dst).reshape(NW, NPAD, 1)

    sp0 = _sc_edge_call(ad0, as0, c0, dst, src).reshape(2, SPAD, D)
    h1, ad1, as1 = _upd0_call(h0, sp0, cnt, l0_mw2.T, r(l0_mb2),
                              l0_uw1[:, :D].T, l0_uw1[:, D:].T, r(l0_ub1),
                              l0_uw2.T, r(l0_ub2), r(l0_lg), r(l0_lb),
                              wd1t, ws1t)

    sp1 = _sc_edge_call(ad1, as1, c1, dst, src).reshape(2, SPAD, D)
    out = _upd1_call(h1, sp1, cnt, l1_mw2.T, r(l1_mb2),
                     l1_uw1[:, :D].T, l1_uw1[:, D:].T, r(l1_ub1),
                     l1_uw2.T, r(l1_ub2), r(l1_lg), r(l1_lb),
                     head_w.T, r(head_b))
    return out


# speedup vs baseline: 1.7299x; 1.7299x over previous
import functools

import jax
import jax.numpy as jnp
from jax import lax
from jax.experimental import pallas as pl
from jax.experimental.pallas import tpu as pltpu
from jax.experimental.pallas import tpu_sc as plsc

N = 10000
E = 320000
D = 128
ED = 16
EPS = 1e-5
HI = lax.Precision.HIGHEST
F32 = jnp.float32

K = 64
KC = 128
NCHUNK = E // K
NCHUNKC = E // KC
NW = 32
TMAX = -(-NCHUNK // NW)
TMAXC = -(-NCHUNKC // NW)
NT = 16
NPAD = 10240
SPAD = 10112
SRPT = SPAD // NT
SQ = (64,) * 9 + (56,)


def _dot(a, b):
    return jnp.dot(a, b, preferred_element_type=F32, precision=HI)


def _ln(h, g, b):
    m = jnp.mean(h, axis=-1, keepdims=True)
    c = h - m
    v = jnp.mean(c * c, axis=-1, keepdims=True)
    return c * lax.rsqrt(v + EPS) * g + b


BN = 1000

def _enc_body(x_ref, ewt, eb, eg, ebeta, wdt, wst, h_ref, ad_ref, as_ref):
    h = jnp.maximum(_dot(x_ref[...], ewt[...]) + eb[...], 0.0)
    h = _ln(h, eg[...], ebeta[...])
    h_ref[...] = h
    ad_ref[...] = _dot(h, wdt[...])
    as_ref[...] = _dot(h, wst[...])


_full = lambda shape: pl.BlockSpec(shape, lambda i: (0,) * len(shape))
_rows = lambda shape: pl.BlockSpec(shape, lambda i: (i,) + (0,) * (len(shape) - 1))

_enc_call = pl.pallas_call(
    _enc_body,
    grid=(N // BN,),
    in_specs=[_rows((BN, D)), _full((D, D)), _full((1, D)), _full((1, D)),
              _full((1, D)), _full((D, D)), _full((D, D))],
    out_specs=[_rows((BN, D))] * 3,
    out_shape=[jax.ShapeDtypeStruct((N, D), F32)] * 3,
)

BE = 2000

def _c_body(ea_ref, w0t, b0, w1t, b1, c0_ref, c1_ref):
    ea = ea_ref[...]
    c0_ref[...] = _dot(ea, w0t[...]) + b0[...]
    c1_ref[...] = _dot(ea, w1t[...]) + b1[...]


_c_call = pl.pallas_call(
    _c_body,
    grid=(E // BE,),
    in_specs=[_rows((BE, ED)), _full((ED, D)), _full((1, D)),
              _full((ED, D)), _full((1, D))],
    out_specs=[_rows((BE, D))] * 2,
    out_shape=[jax.ShapeDtypeStruct((E, D), F32)] * 2,
)

def _aggr_update(h, sp, cntp, w2t, mb2, u1at, u1bt, ub1, uw2t, ub2, lg, lb):
    s = sp[0] + sp[1]
    cnt = jnp.sum(cntp, axis=0)
    aggr = _dot(s, w2t) / jnp.maximum(cnt, 1.0)
    aggr = aggr + jnp.where(cnt > 0, 1.0, 0.0) * mb2
    u = jnp.maximum(_dot(h, u1at) + _dot(aggr, u1bt) + ub1, 0.0)
    u = _dot(u, uw2t) + ub2
    return _ln(h + u, lg, lb)


def _upd0_body(h_ref, sp_ref, cnt_ref, w2t, mb2, u1at, u1bt, ub1, uw2t, ub2,
               lg, lb, wdt, wst, h1_ref, ad_ref, as_ref):
    hn = _aggr_update(h_ref[...], sp_ref[...], cnt_ref[...], w2t[...],
                      mb2[...], u1at[...], u1bt[...], ub1[...], uw2t[...],
                      ub2[...], lg[...], lb[...])
    h1_ref[...] = hn
    ad_ref[...] = _dot(hn, wdt[...])
    as_ref[...] = _dot(hn, wst[...])


def _upd1_body(h_ref, sp_ref, cnt_ref, w2t, mb2, u1at, u1bt, ub1, uw2t, ub2,
               lg, lb, hwt, hb, out_ref):
    hn = _aggr_update(h_ref[...], sp_ref[...], cnt_ref[...], w2t[...],
                      mb2[...], u1at[...], u1bt[...], ub1[...], uw2t[...],
                      ub2[...], lg[...], lb[...])
    out_ref[...] = _dot(hn, hwt[...]) + hb[...]


_upd_common_specs = [
    _rows((BN, D)),
    pl.BlockSpec((2, BN, D), lambda i: (0, i, 0)),
    pl.BlockSpec((NW, BN, 1), lambda i: (0, i, 0)),
    _full((D, D)), _full((1, D)),
    _full((D, D)), _full((D, D)), _full((1, D)),
    _full((D, D)), _full((1, D)),
    _full((1, D)), _full((1, D)),
]

_upd0_call = pl.pallas_call(
    _upd0_body,
    grid=(N // BN,),
    in_specs=_upd_common_specs + [_full((D, D)), _full((D, D))],
    out_specs=[_rows((BN, D))] * 3,
    out_shape=[jax.ShapeDtypeStruct((N, D), F32)] * 3,
)

_upd1_call = pl.pallas_call(
    _upd1_body,
    grid=(N // BN,),
    in_specs=_upd_common_specs + [_full((D, D)), _full((1, D))],
    out_specs=_rows((BN, D)),
    out_shape=jax.ShapeDtypeStruct((N, D), F32),
)


def _sc_cnt_body(dst_hbm, cnt_hbm, dsti, cntloc):
    cid = lax.axis_index("c")
    sid = lax.axis_index("s")
    wid = sid * 2 + cid

    zero16 = jnp.zeros((16,), F32)
    ones16 = jnp.ones((16,), F32)

    def zbody(j, carry):
        for q in range(NPAD // KC // 16):
            cntloc[pl.ds(j * (NPAD // KC) + q * 16, 16)] = zero16
        return carry

    lax.fori_loop(0, KC, zbody, 0)

    def body(t, carry):
        chunk = wid + t * NW

        @pl.when(chunk < NCHUNKC)
        def _():
            pltpu.sync_copy(dst_hbm.at[pl.ds(chunk * KC, KC)], dsti)
            for u in range(KC // 16):
                plsc.addupdate_scatter(cntloc, [dsti[pl.ds(u * 16, 16)]], ones16)
        return carry

    lax.fori_loop(0, TMAXC, body, 0)
    pltpu.sync_copy(cntloc, cnt_hbm.at[pl.ds(wid * NPAD, NPAD)])


_sc_cnt_call = functools.partial(
    pl.kernel,
    out_type=jax.ShapeDtypeStruct((NW * NPAD,), F32),
    mesh=plsc.VectorSubcoreMesh(core_axis_name="c", subcore_axis_name="s"),
    compiler_params=pltpu.CompilerParams(needs_layout_passes=False),
    scratch_types=[
        pltpu.VMEM((KC,), jnp.int32),
        pltpu.VMEM((NPAD,), F32),
    ],
)(_sc_cnt_body)


def _sc_edge_body(ad_hbm, as_hbm, c_hbm, dst_hbm, src_hbm, out_hbm,
                  dstq, srcq, adg0, adg1, asg0, asg1, cbuf0, cbuf1, s_sh,
                  sa0, sa1, ss0, ss1, sc0, sc1):
    cid = lax.axis_index("c")
    sid = lax.axis_index("s")
    wid = sid * 2 + cid
    nsb = NCHUNK // 2
    slo = wid * (nsb // NW) + jnp.minimum(wid, nsb % NW)
    lo = 2 * slo
    cnt = 2 * (nsb // NW) + jnp.where(wid < nsb % NW, 2, 0)

    adg = (adg0, adg1)
    asg = (asg0, asg1)
    cbuf = (cbuf0, cbuf1)
    sa = (sa0, sa1)
    ss = (ss0, ss1)
    sc = (sc0, sc1)

    zero16 = jnp.zeros((16,), F32)

    def zbody(j, carry):
        for g in range(D // 16):
            cbuf0[j, pl.ds(g * 16, 16)] = zero16
        return carry

    lax.fori_loop(0, K, zbody, 0)
    off = 0
    for q in SQ:
        pltpu.sync_copy(cbuf0.at[pl.ds(0, q)],
                        s_sh.at[pl.ds(sid * SRPT + off, q)])
        off += q
    plsc.subcore_barrier()

    def loadidx(p, u):
        sb = slo + u

        @pl.when(sb < NCHUNK // 2)
        def _():
            pltpu.sync_copy(dst_hbm.at[sb], dstq.at[p])
            pltpu.sync_copy(src_hbm.at[sb], srcq.at[p])

    def issue(m, b, p, i):
        @pl.when(m < cnt)
        def _():
            e0 = (lo + m) * K
            pltpu.async_copy(ad_hbm.at[dstq.at[p, i]], adg[b], sa[b])
            pltpu.async_copy(as_hbm.at[srcq.at[p, i]], asg[b], ss[b])
            pltpu.async_copy(c_hbm.at[pl.ds(e0, K)], cbuf[b], sc[b])

    def process(m, b, p, i):
        @pl.when(m < cnt)
        def _():
            pltpu.make_async_copy(ad_hbm.at[dstq.at[p, i]], adg[b], sa[b]).wait()
            pltpu.make_async_copy(as_hbm.at[srcq.at[p, i]], asg[b], ss[b]).wait()
            pltpu.make_async_copy(c_hbm.at[pl.ds(0, K)], cbuf[b], sc[b]).wait()

            def jbody(j, c2):
                for g in range(D // 16):
                    s = pl.ds(g * 16, 16)
                    cbuf[b][j, s] = jnp.maximum(
                        cbuf[b][j, s] + adg[b][j, s] + asg[b][j, s], 0.0)
                return c2

            lax.fori_loop(0, K, jbody, 0)
            pltpu.sync_copy(cbuf[b], s_sh.at[dstq.at[p, i]], add=True)

    loadidx(0, 0)
    issue(0, 0, 0, 0)

    def body(u, carry):
        p = lax.rem(u, 2)
        pn = 1 - p
        m0 = 2 * u
        issue(m0 + 1, 1, p, 1)
        loadidx(pn, u + 1)
        process(m0, 0, p, 0)
        issue(m0 + 2, 0, pn, 0)
        process(m0 + 1, 1, p, 1)
        return carry

    lax.fori_loop(0, (TMAX + 1) // 2, body, 0)
    plsc.subcore_barrier()

    off = 0
    for q in SQ:
        pltpu.sync_copy(s_sh.at[pl.ds(sid * SRPT + off, q)],
                        cbuf0.at[pl.ds(0, q)])
        pltpu.sync_copy(cbuf0.at[pl.ds(0, q)],
                        out_hbm.at[pl.ds(cid * SPAD + sid * SRPT + off, q)])
        off += q


_sc_edge_call = functools.partial(
    pl.kernel,
    out_type=jax.ShapeDtypeStruct((2 * SPAD, D), F32),
    mesh=plsc.VectorSubcoreMesh(core_axis_name="c", subcore_axis_name="s"),
    compiler_params=pltpu.CompilerParams(needs_layout_passes=False),
    scratch_types=[
        pltpu.VMEM((2, 2, K), jnp.int32),
        pltpu.VMEM((2, 2, K), jnp.int32),
        pltpu.VMEM((K, D), F32),
        pltpu.VMEM((K, D), F32),
        pltpu.VMEM((K, D), F32),
        pltpu.VMEM((K, D), F32),
        pltpu.VMEM((K, D), F32),
        pltpu.VMEM((K, D), F32),
        pltpu.VMEM_SHARED((SPAD, D), F32),
        pltpu.SemaphoreType.DMA,
        pltpu.SemaphoreType.DMA,
        pltpu.SemaphoreType.DMA,
        pltpu.SemaphoreType.DMA,
        pltpu.SemaphoreType.DMA,
        pltpu.SemaphoreType.DMA,
    ],
)(_sc_edge_body)


def kernel(x, edge_index, edge_attr, enc_w, enc_b, enc_g, enc_beta,
           l0_mw1, l0_mb1, l0_mw2, l0_mb2, l0_uw1, l0_ub1, l0_uw2, l0_ub2, l0_lg, l0_lb,
           l1_mw1, l1_mb1, l1_mw2, l1_mb2, l1_uw1, l1_ub1, l1_uw2, l1_ub2, l1_lg, l1_lb,
           head_w, head_b):
    src = edge_index[0]
    dst = edge_index[1]
    r = lambda v: v.reshape(1, D)

    wd0t = l0_mw1[:, :D].T
    ws0t = l0_mw1[:, D:2 * D].T
    we0t = l0_mw1[:, 2 * D:].T
    wd1t = l1_mw1[:, :D].T
    ws1t = l1_mw1[:, D:2 * D].T
    we1t = l1_mw1[:, 2 * D:].T

    h0, ad0, as0 = _enc_call(x, enc_w.T, r(enc_b), r(enc_g), r(enc_beta),
                             wd0t, ws0t)
    c0, c1 = _c_call(edge_attr, we0t, r(l0_mb1), we1t, r(l1_mb1))
    cnt = _sc_cnt_call(dst).reshape(NW, NPAD, 1)

    sp0 = _sc_edge_call(ad0, as0, c0, dst.reshape(NCHUNK // 2, 2, K), src.reshape(NCHUNK // 2, 2, K)).reshape(2, SPAD, D)
    h1, ad1, as1 = _upd0_call(h0, sp0, cnt, l0_mw2.T, r(l0_mb2),
                              l0_uw1[:, :D].T, l0_uw1[:, D:].T, r(l0_ub1),
                              l0_uw2.T, r(l0_ub2), r(l0_lg), r(l0_lb),
                              wd1t, ws1t)

    sp1 = _sc_edge_call(ad1, as1, c1, dst.reshape(NCHUNK // 2, 2, K), src.reshape(NCHUNK // 2, 2, K)).reshape(2, SPAD, D)
    out = _upd1_call(h1, sp1, cnt, l1_mw2.T, r(l1_mb2),
                     l1_uw1[:, :D].T, l1_uw1[:, D:].T, r(l1_ub1),
                     l1_uw2.T, r(l1_ub2), r(l1_lg), r(l1_lb),
                     head_w.T, r(head_b))
    return out
